# baseline (speedup 1.0000x reference)
import functools

import jax
import jax.numpy as jnp
from jax import lax
from jax.experimental import pallas as pl
from jax.experimental.pallas import tpu as pltpu
from jax.experimental.pallas import tpu_sc as plsc

_B = 4096
_DIC = 100
_D = 64
_NE = 6
_NPOS = _B * _DIC
_DP = _DIC // 2
_NPAIR = _B * _DP
_NC = 2
_NS = 16
_NW = _NC * _NS
_PER_W = _NPAIR // _NW
_CH = 128
_NCH = _PER_W // _CH
_NB = 2


def _tab_body(nm2_ref, vt_ref, out_ref):
    nm2 = nm2_ref[...]
    zero = jnp.zeros((_D,), jnp.float32)
    for e0 in range(_NE + 1):
        left = vt_ref[e0] if e0 < _NE else zero
        for e1 in range(_NE + 1):
            right = vt_ref[e1] if e1 < _NE else zero
            out_ref[e0, e1] = nm2 + jnp.concatenate([left, right], axis=-1)


def _build_table(name_embedding, value_table):
    out = pl.pallas_call(
        _tab_body,
        out_shape=jax.ShapeDtypeStruct((_NE + 1, _NE + 1, _DP, 2 * _D), jnp.float32),
    )(name_embedding.reshape(_DP, 2 * _D), value_table)
    return out.reshape((_NE + 1) * (_NE + 1) * _DP, 2 * _D)


def _sc_body(ctab_h, xc_h, ys_h, dpl_h, out_h,
             xc_v, ys_v, dp_v, i_v, stage_v, sem_g, sem_s):
    wid = lax.axis_index("s") * _NC + lax.axis_index("c")
    base0 = wid * _PER_W
    pltpu.sync_copy(xc_h.at[pl.ds(base0, _PER_W)], xc_v)
    pltpu.sync_copy(ys_h.at[pl.ds(base0, _PER_W)], ys_v)
    pltpu.sync_copy(dpl_h, dp_v)

    def idx_chunk(c, carry):
        for j in range(_CH // 16):
            sl = pl.ds(c * _CH + j * 16, 16)
            xc = xc_v[sl]
            dp = dp_v[sl]
            yv = ys_v[sl]
            xe = xc & 7
            xo = xc >> 3
            d0 = dp * 2
            e0 = jnp.where(d0 == yv, _NE, xe)
            e1 = jnp.where(d0 + 1 == yv, _NE, xo)
            i_v[c, pl.ds(j * 16, 16)] = (e0 * (_NE + 1) + e1) * _DP + dp
        return carry

    lax.fori_loop(0, _NCH, idx_chunk, 0)

    def start_gather(c, b):
        pltpu.async_copy(ctab_h.at[i_v.at[c]], stage_v.at[b], sem_g)

    def wait_gather(c, b):
        pltpu.make_async_copy(ctab_h.at[i_v.at[c]], stage_v.at[b], sem_g).wait()

    for b in range(_NB):
        start_gather(b, b)

    def outer(t, carry):
        c0 = t * _NB
        for b in range(_NB):
            c = c0 + b
            base = base0 + c * _CH
            wait_gather(c, b)
            pltpu.async_copy(stage_v.at[b], out_h.at[pl.ds(base, _CH)], sem_s)
            pltpu.make_async_copy(
                stage_v.at[b], out_h.at[pl.ds(base, _CH)], sem_s).wait()

            @pl.when(c + _NB < _NCH)
            def _():
                start_gather(c + _NB, b)
        return carry

    lax.fori_loop(0, _NCH // _NB, outer, 0)


def _sc_gather(ctab, xc, ys, dpl):
    mesh = plsc.VectorSubcoreMesh(core_axis_name="c", subcore_axis_name="s")
    run = functools.partial(
        pl.kernel,
        out_type=jax.ShapeDtypeStruct((_NPAIR, 2 * _D), jnp.float32),
        mesh=mesh,
        scratch_types=[
            pltpu.VMEM((_PER_W,), jnp.int32),
            pltpu.VMEM((_PER_W,), jnp.int32),
            pltpu.VMEM((_PER_W,), jnp.int32),
            pltpu.VMEM((_NCH, _CH), jnp.int32),
            pltpu.VMEM((_NB, _CH, 2 * _D), jnp.float32),
            pltpu.SemaphoreType.DMA,
            pltpu.SemaphoreType.DMA,
        ],
    )(_sc_body)
    return run(ctab, xc, ys, dpl)


@jax.jit
def kernel(x, y, name_embedding, value_table):
    x = x.astype(jnp.int32)
    y = y.astype(jnp.int32)
    ctab = _build_table(name_embedding, value_table)
    pair_w = jnp.array([1, 8], dtype=jnp.int32)
    xc = jnp.sum(x.reshape(_NPAIR, 2) * pair_w[None, :], axis=1).reshape(_NPAIR)
    ys = jnp.repeat(y, _DP)
    dpl = jnp.tile(jnp.arange(_DP, dtype=jnp.int32), _PER_W // _DP)
    out = _sc_gather(ctab, xc, ys, dpl)
    return out.reshape(_B, _DIC, _D)
